# baseline (speedup 1.0000x reference)
import jax
import jax.numpy as jnp
from jax.experimental import pallas as pl
from jax.experimental.pallas import tpu as pltpu

N = 4096
BR = 512
NB = N // BR


def _gcn_kernel(x_ref, adj_ref, w1_ref, b1_ref, w3_ref, b3_ref, wg_ref,
                bg_ref, out_ref, deg_ref, v_ref):
    i = pl.program_id(0)
    ib = jax.lax.rem(i, NB)

    adj_blk = adj_ref[...]
    row_ids = jax.lax.broadcasted_iota(jnp.int32, (BR, N), 0) + ib * BR
    col_ids = jax.lax.broadcasted_iota(jnp.int32, (BR, N), 1)
    a_hat = jnp.maximum(adj_blk, (row_ids == col_ids).astype(jnp.float32))

    @pl.when(i == 0)
    def _init_deg():
        deg_ref[...] = jnp.zeros_like(deg_ref)

    @pl.when(i < NB)
    def _phase0():
        ones = jnp.ones((BR, 1), dtype=jnp.float32)
        deg_ref[...] += jax.lax.dot_general(
            a_hat, ones, (((0,), (0,)), ((), ())),
            preferred_element_type=jnp.float32)

    @pl.when(i == NB)
    def _transition():
        deg = deg_ref[...]
        dinv = jnp.where(deg > 0, jax.lax.rsqrt(deg), 0.0)
        deg_ref[...] = dinv
        h = jax.nn.relu(
            jnp.dot(x_ref[...], w1_ref[...],
                    preferred_element_type=jnp.float32) + b1_ref[...])
        h = jax.nn.relu(
            jnp.dot(h, w3_ref[...],
                    preferred_element_type=jnp.float32) + b3_ref[...])
        hw = jnp.dot(h, wg_ref[...], preferred_element_type=jnp.float32)
        v_ref[...] = dinv * hw
        out_ref[...] = jnp.zeros_like(out_ref)

    @pl.when(i >= NB)
    def _phase1():
        v_blk = v_ref[pl.ds(ib * BR, BR), :]
        out_ref[...] += jax.lax.dot_general(
            a_hat, v_blk, (((0,), (0,)), ((), ())),
            preferred_element_type=jnp.float32)

    @pl.when(i == 2 * NB - 1)
    def _finalize():
        out_ref[...] = deg_ref[...] * out_ref[...] + bg_ref[...]


def kernel(x, adj, W1, b1, W3, b3, Wg, bg):
    b1r = b1.reshape(1, 16)
    b3r = b3.reshape(1, 3)
    bgr = bg.reshape(1, 3)
    out = pl.pallas_call(
        _gcn_kernel,
        grid=(2 * NB,),
        in_specs=[
            pl.BlockSpec((N, 3), lambda i: (0, 0)),
            pl.BlockSpec((BR, N), lambda i: (i % NB, 0)),
            pl.BlockSpec((3, 16), lambda i: (0, 0)),
            pl.BlockSpec((1, 16), lambda i: (0, 0)),
            pl.BlockSpec((16, 3), lambda i: (0, 0)),
            pl.BlockSpec((1, 3), lambda i: (0, 0)),
            pl.BlockSpec((3, 3), lambda i: (0, 0)),
            pl.BlockSpec((1, 3), lambda i: (0, 0)),
        ],
        out_specs=pl.BlockSpec((N, 3), lambda i: (0, 0)),
        out_shape=jax.ShapeDtypeStruct((N, 3), jnp.float32),
        scratch_shapes=[
            pltpu.VMEM((N, 1), jnp.float32),
            pltpu.VMEM((N, 3), jnp.float32),
        ],
        compiler_params=pltpu.CompilerParams(
            dimension_semantics=("arbitrary",)),
    )(x, adj, W1, b1r, W3, b3r, Wg, bgr)
    return out


# speedup vs baseline: 1.0842x; 1.0842x over previous
import jax
import jax.numpy as jnp
from jax.experimental import pallas as pl
from jax.experimental.pallas import tpu as pltpu

N = 4096
BR = 512
NB = N // BR


def _gcn_kernel(x_ref, adj_ref, sq_ref, w1_ref, b1_ref, w3_ref, b3_ref,
                wg_ref, bg_ref, out_ref, deg_ref, miss_ref, v_ref, vb_ref):
    i = pl.program_id(0)
    ib = jax.lax.rem(i, NB)

    adj_b = adj_ref[...].astype(jnp.bfloat16)

    @pl.when(i == 0)
    def _init():
        deg_ref[...] = jnp.zeros_like(deg_ref)

    @pl.when(i < NB)
    def _phase0():
        ones = jnp.ones((BR, 1), dtype=jnp.bfloat16)
        deg_ref[...] += jax.lax.dot_general(
            adj_b, ones, (((0,), (0,)), ((), ())),
            preferred_element_type=jnp.float32)
        sq = sq_ref[...]
        r_ids = jax.lax.broadcasted_iota(jnp.int32, (BR, BR), 0)
        c_ids = jax.lax.broadcasted_iota(jnp.int32, (BR, BR), 1)
        eye = (r_ids == c_ids).astype(jnp.float32)
        diag = jnp.sum(sq * eye, axis=1, keepdims=True)
        miss_ref[pl.ds(ib * BR, BR), :] = jnp.where(diag > 0, 0.0, 1.0)

    @pl.when(i == NB)
    def _transition():
        deg = deg_ref[...] + miss_ref[...]
        dinv = jax.lax.rsqrt(jnp.maximum(deg, 1.0))
        deg_ref[...] = dinv
        h = jax.nn.relu(
            jnp.dot(x_ref[...], w1_ref[...],
                    preferred_element_type=jnp.float32) + b1_ref[...])
        h = jax.nn.relu(
            jnp.dot(h, w3_ref[...],
                    preferred_element_type=jnp.float32) + b3_ref[...])
        hw = jnp.dot(h, wg_ref[...], preferred_element_type=jnp.float32)
        v = dinv * hw
        v_ref[...] = v
        vb_ref[...] = v.astype(jnp.bfloat16)
        out_ref[...] = jnp.zeros_like(out_ref)

    @pl.when(i >= NB)
    def _phase1():
        v_blk = vb_ref[pl.ds(ib * BR, BR), :]
        out_ref[...] += jax.lax.dot_general(
            adj_b, v_blk, (((0,), (0,)), ((), ())),
            preferred_element_type=jnp.float32)

    @pl.when(i == 2 * NB - 1)
    def _finalize():
        out_ref[...] = (deg_ref[...] * (out_ref[...]
                                        + miss_ref[...] * v_ref[...])
                        + bg_ref[...])


def kernel(x, adj, W1, b1, W3, b3, Wg, bg):
    b1r = b1.reshape(1, 16)
    b3r = b3.reshape(1, 3)
    bgr = bg.reshape(1, 3)
    out = pl.pallas_call(
        _gcn_kernel,
        grid=(2 * NB,),
        in_specs=[
            pl.BlockSpec((N, 3), lambda i: (0, 0)),
            pl.BlockSpec((BR, N), lambda i: (i % NB, 0)),
            pl.BlockSpec((BR, BR), lambda i: (i % NB, i % NB)),
            pl.BlockSpec((3, 16), lambda i: (0, 0)),
            pl.BlockSpec((1, 16), lambda i: (0, 0)),
            pl.BlockSpec((16, 3), lambda i: (0, 0)),
            pl.BlockSpec((1, 3), lambda i: (0, 0)),
            pl.BlockSpec((3, 3), lambda i: (0, 0)),
            pl.BlockSpec((1, 3), lambda i: (0, 0)),
        ],
        out_specs=pl.BlockSpec((N, 3), lambda i: (0, 0)),
        out_shape=jax.ShapeDtypeStruct((N, 3), jnp.float32),
        scratch_shapes=[
            pltpu.VMEM((N, 1), jnp.float32),
            pltpu.VMEM((N, 1), jnp.float32),
            pltpu.VMEM((N, 3), jnp.float32),
            pltpu.VMEM((N, 3), jnp.bfloat16),
        ],
        compiler_params=pltpu.CompilerParams(
            dimension_semantics=("arbitrary",)),
    )(x, adj, adj, W1, b1r, W3, b3r, Wg, bgr)
    return out


# speedup vs baseline: 1.1236x; 1.0364x over previous
import jax
import jax.numpy as jnp
from jax.experimental import pallas as pl
from jax.experimental.pallas import tpu as pltpu

N = 4096
BR = 512
NB = N // BR


def _gcn_kernel(x_ref, adj_ref, sq_ref, w1_ref, b1_ref, w3_ref, b3_ref,
                wg_ref, bg_ref, out_ref, deg_ref, miss_ref, acc_ref,
                vb_ref, vt_ref):
    i = pl.program_id(0)
    ib = jax.lax.rem(i, NB)

    adj_b = adj_ref[...].astype(jnp.bfloat16)

    @pl.when(i == 0)
    def _init():
        deg_ref[...] = jnp.zeros_like(deg_ref)

    @pl.when(i < NB)
    def _phase0():
        ones = jnp.ones((BR, 1), dtype=jnp.bfloat16)
        deg_ref[...] += jax.lax.dot_general(
            ones, adj_b, (((0,), (0,)), ((), ())),
            preferred_element_type=jnp.float32)
        sq = sq_ref[...]
        r_ids = jax.lax.broadcasted_iota(jnp.int32, (BR, BR), 0)
        c_ids = jax.lax.broadcasted_iota(jnp.int32, (BR, BR), 1)
        eye = (r_ids == c_ids).astype(jnp.float32)
        diag = jnp.sum(sq * eye, axis=1, keepdims=True)
        miss_ref[pl.ds(ib * BR, BR), :] = jnp.where(diag > 0, 0.0, 1.0)

    @pl.when(i == NB)
    def _transition():
        miss_row = jnp.transpose(miss_ref[...], (1, 0))
        deg = deg_ref[...] + miss_row
        dinv_row = jax.lax.rsqrt(jnp.maximum(deg, 1.0))
        deg_ref[...] = dinv_row
        h = jax.nn.relu(
            jnp.dot(x_ref[...], w1_ref[...],
                    preferred_element_type=jnp.float32) + b1_ref[...])
        h = jax.nn.relu(
            jnp.dot(h, w3_ref[...],
                    preferred_element_type=jnp.float32) + b3_ref[...])
        hw = jnp.dot(h, wg_ref[...], preferred_element_type=jnp.float32)
        dinv_col = jnp.transpose(dinv_row, (1, 0))
        v = dinv_col * hw
        vb_ref[...] = v.astype(jnp.bfloat16)
        vt_ref[...] = jnp.transpose(v, (1, 0))
        acc_ref[...] = jnp.zeros_like(acc_ref)

    @pl.when(i >= NB)
    def _phase1():
        v_blk = vb_ref[pl.ds(ib * BR, BR), :]
        acc_ref[...] += jax.lax.dot_general(
            v_blk, adj_b, (((0,), (0,)), ((), ())),
            preferred_element_type=jnp.float32)

    @pl.when(i == 2 * NB - 1)
    def _finalize():
        miss_row = jnp.transpose(miss_ref[...], (1, 0))
        out_t = deg_ref[...] * (acc_ref[...] + miss_row * vt_ref[...])
        out_ref[...] = jnp.transpose(out_t, (1, 0)) + bg_ref[...]


def kernel(x, adj, W1, b1, W3, b3, Wg, bg):
    b1r = b1.reshape(1, 16)
    b3r = b3.reshape(1, 3)
    bgr = bg.reshape(1, 3)
    out = pl.pallas_call(
        _gcn_kernel,
        grid=(2 * NB,),
        in_specs=[
            pl.BlockSpec((N, 3), lambda i: (0, 0)),
            pl.BlockSpec((BR, N), lambda i: (i % NB, 0)),
            pl.BlockSpec((BR, BR), lambda i: (i % NB, i % NB)),
            pl.BlockSpec((3, 16), lambda i: (0, 0)),
            pl.BlockSpec((1, 16), lambda i: (0, 0)),
            pl.BlockSpec((16, 3), lambda i: (0, 0)),
            pl.BlockSpec((1, 3), lambda i: (0, 0)),
            pl.BlockSpec((3, 3), lambda i: (0, 0)),
            pl.BlockSpec((1, 3), lambda i: (0, 0)),
        ],
        out_specs=pl.BlockSpec((N, 3), lambda i: (0, 0)),
        out_shape=jax.ShapeDtypeStruct((N, 3), jnp.float32),
        scratch_shapes=[
            pltpu.VMEM((1, N), jnp.float32),
            pltpu.VMEM((N, 1), jnp.float32),
            pltpu.VMEM((3, N), jnp.float32),
            pltpu.VMEM((N, 3), jnp.bfloat16),
            pltpu.VMEM((3, N), jnp.float32),
        ],
        compiler_params=pltpu.CompilerParams(
            dimension_semantics=("arbitrary",)),
    )(x, adj, adj, W1, b1r, W3, b3r, Wg, bgr)
    return out


# speedup vs baseline: 1.6121x; 1.4347x over previous
import jax
import jax.numpy as jnp
from jax.experimental import pallas as pl
from jax.experimental.pallas import tpu as pltpu

N = 4096
BR = 256
NB = N // BR


def _gcn_kernel(x_ref, adj_ref, sq_ref, w1_ref, b1_ref, w3_ref, b3_ref,
                wg_ref, bg_ref, out_ref, deg_ref, miss_ref, cache_ref):
    i = pl.program_id(0)

    @pl.when(i == 0)
    def _init():
        deg_ref[...] = jnp.zeros_like(deg_ref)

    adj_b = adj_ref[...].astype(jnp.bfloat16)
    cache_ref[pl.ds(i * BR, BR), :] = adj_b
    ones = jnp.ones((BR, 1), dtype=jnp.bfloat16)
    deg_ref[...] += jax.lax.dot_general(
        ones, adj_b, (((0,), (0,)), ((), ())),
        preferred_element_type=jnp.float32)
    sq = sq_ref[...]
    r_ids = jax.lax.broadcasted_iota(jnp.int32, (BR, BR), 0)
    c_ids = jax.lax.broadcasted_iota(jnp.int32, (BR, BR), 1)
    eye = (r_ids == c_ids).astype(jnp.float32)
    diag = jnp.sum(sq * eye, axis=1, keepdims=True)
    miss_ref[pl.ds(i * BR, BR), :] = jnp.where(diag > 0, 0.0, 1.0)

    @pl.when(i == NB - 1)
    def _finalize():
        miss_row = jnp.transpose(miss_ref[...], (1, 0))
        deg = deg_ref[...] + miss_row
        dinv_row = jax.lax.rsqrt(jnp.maximum(deg, 1.0))
        h = jax.nn.relu(
            jnp.dot(x_ref[...], w1_ref[...],
                    preferred_element_type=jnp.float32) + b1_ref[...])
        h = jax.nn.relu(
            jnp.dot(h, w3_ref[...],
                    preferred_element_type=jnp.float32) + b3_ref[...])
        hw = jnp.dot(h, wg_ref[...], preferred_element_type=jnp.float32)
        dinv_col = jnp.transpose(dinv_row, (1, 0))
        v = dinv_col * hw
        vb = v.astype(jnp.bfloat16)
        vt = jnp.transpose(v, (1, 0))
        acc = jnp.zeros((3, N), dtype=jnp.float32)
        for j in range(NB):
            blk = cache_ref[j * BR:(j + 1) * BR, :]
            vblk = vb[j * BR:(j + 1) * BR, :]
            acc = acc + jax.lax.dot_general(
                vblk, blk, (((0,), (0,)), ((), ())),
                preferred_element_type=jnp.float32)
        out_t = dinv_row * (acc + miss_row * vt)
        out_ref[...] = jnp.transpose(out_t, (1, 0)) + bg_ref[...]


def kernel(x, adj, W1, b1, W3, b3, Wg, bg):
    b1r = b1.reshape(1, 16)
    b3r = b3.reshape(1, 3)
    bgr = bg.reshape(1, 3)
    out = pl.pallas_call(
        _gcn_kernel,
        grid=(NB,),
        in_specs=[
            pl.BlockSpec((N, 3), lambda i: (0, 0)),
            pl.BlockSpec((BR, N), lambda i: (i, 0)),
            pl.BlockSpec((BR, BR), lambda i: (i, i)),
            pl.BlockSpec((3, 16), lambda i: (0, 0)),
            pl.BlockSpec((1, 16), lambda i: (0, 0)),
            pl.BlockSpec((16, 3), lambda i: (0, 0)),
            pl.BlockSpec((1, 3), lambda i: (0, 0)),
            pl.BlockSpec((3, 3), lambda i: (0, 0)),
            pl.BlockSpec((1, 3), lambda i: (0, 0)),
        ],
        out_specs=pl.BlockSpec((N, 3), lambda i: (0, 0)),
        out_shape=jax.ShapeDtypeStruct((N, 3), jnp.float32),
        scratch_shapes=[
            pltpu.VMEM((1, N), jnp.float32),
            pltpu.VMEM((N, 1), jnp.float32),
            pltpu.VMEM((N, N), jnp.bfloat16),
        ],
        compiler_params=pltpu.CompilerParams(
            dimension_semantics=("arbitrary",)),
    )(x, adj, adj, W1, b1r, W3, b3r, Wg, bgr)
    return out
